# baseline (speedup 1.0000x reference)
import jax
import jax.numpy as jnp
from jax import lax
from jax.experimental import pallas as pl
from jax.experimental.pallas import tpu as pltpu
from jax.experimental.pallas import tpu_sc as plsc

_NB = 64
_GRID = _NB * _NB * _NB
_B = 16
_P = 131072
_NC = 2
_NS = 16
_L = 16
_ROUNDS = _B // _NC
_PPT = _P // _NS
_FPT = _PPT * 3
_NVEC = _FPT // _L
_GSLICE = _GRID // _NS
_CHUNK = 128
_NCHUNK = _PPT // _CHUNK
_RING = 4
_MAXOFF = 2 * _GRID
_NSUP1 = _NCHUNK // _RING - 1
_MB = (_NVEC // 4) // _NSUP1


def _body(x_hbm, out_hbm, pts0, pts1, idx0, idx1, idx2, idx3, ones, zeros,
          maxv, allmax, shared, sem0, sem1, sem2, sem3, psem0, psem1):
    idxs = (idx0, idx1, idx2, idx3)
    sems = (sem0, sem1, sem2, sem3)
    ptsb = (pts0, pts1)
    psems = (psem0, psem1)
    c = lax.axis_index("c")
    s = lax.axis_index("s")

    for k in range(_CHUNK // _L):
        ones[pl.ds(k * _L, _L)] = jnp.ones((_L,), jnp.float32)

    def zero_body(i, _):
        zeros[pl.ds(i * _L, _L)] = jnp.zeros((_L,), jnp.float32)
        return 0
    lax.fori_loop(0, _GSLICE // _L, zero_body, 0)

    pltpu.sync_copy(zeros, shared.at[pl.ds(s * _GSLICE, _GSLICE)])
    pltpu.sync_copy(zeros, shared.at[pl.ds(_GRID + s * _GSLICE, _GSLICE)])

    def stage(r, par):
        b = c * _ROUNDS + r
        for q in range(3):
            pltpu.async_copy(
                x_hbm.at[pl.ds((3 * b + q) * _P + s * _PPT, _PPT)],
                ptsb[par].at[pl.ds(q * _PPT, _PPT)], psems[par])

    def stage_wait(r, par):
        b = c * _ROUNDS + r
        for q in range(3):
            pltpu.make_async_copy(
                x_hbm.at[pl.ds((3 * b + q) * _P + s * _PPT, _PPT)],
                ptsb[par].at[pl.ds(q * _PPT, _PPT)], psems[par]).wait()

    def local_max(r, par):
        stage_wait(r, par)
        pts = ptsb[par]

        def max_body(i, carry):
            m0, m1, m2, m3 = carry
            base = i * (4 * _L)
            v0 = pts[pl.ds(base, _L)]
            v1 = pts[pl.ds(base + _L, _L)]
            v2 = pts[pl.ds(base + 2 * _L, _L)]
            v3 = pts[pl.ds(base + 3 * _L, _L)]
            return (jnp.maximum(m0, jnp.abs(v0)),
                    jnp.maximum(m1, jnp.abs(v1)),
                    jnp.maximum(m2, jnp.abs(v2)),
                    jnp.maximum(m3, jnp.abs(v3)))
        z4 = jnp.zeros((_L,), jnp.float32)
        m0, m1, m2, m3 = lax.fori_loop(0, _NVEC // 4, max_body,
                                       (z4, z4, z4, z4), unroll=2)
        maxv[...] = jnp.maximum(jnp.maximum(m0, m1), jnp.maximum(m2, m3))
        pltpu.sync_copy(
            maxv,
            shared.at[pl.ds(_MAXOFF + par * _NS * _L + s * _L, _L)])

    def do_round(r, par):
        b = c * _ROUNDS + r
        pts = ptsb[par]
        ptsn = ptsb[1 - par]
        gbase = par * _GRID

        pltpu.sync_copy(
            shared.at[pl.ds(_MAXOFF + par * _NS * _L, _NS * _L)], allmax)

        def gmax_body(i, mm):
            return jnp.maximum(mm, allmax[pl.ds(i * _L, _L)])
        mm = lax.fori_loop(0, _NS, gmax_body, jnp.zeros((_L,), jnp.float32))
        gmax = mm[0]
        for i in range(1, _L):
            gmax = jnp.maximum(gmax, mm[i])
        thick = jnp.maximum(2.0 * gmax, 1e-5)

        def compute_chunk(g, j):
            for v in range(_CHUNK // _L):
                pb = g * _CHUNK + v * _L
                x = pts[pl.ds(pb, _L)]
                y = pts[pl.ds(_PPT + pb, _L)]
                z = pts[pl.ds(2 * _PPT + pb, _L)]
                cx = ((x + gmax) / thick * 64.0).astype(jnp.int32)
                cy = ((y + gmax) / thick * 64.0).astype(jnp.int32)
                cz = ((z + gmax) / thick * 64.0).astype(jnp.int32)
                cx = jnp.clip(cx, 0, _NB - 1)
                cy = jnp.clip(cy, 0, _NB - 1)
                cz = jnp.clip(cz, 0, _NB - 1)
                flat = (cx * _NB + cy) * _NB + (cz + gbase)
                idxs[j][pl.ds(v * _L, _L)] = flat

        for j in range(_RING):
            compute_chunk(jnp.int32(j), j)
            pltpu.async_copy(ones, shared.at[idxs[j]], sems[j])

        rp1 = jnp.where(r + 1 < _ROUNDS, r + 1, r + 1 - _ROUNDS)
        stage_wait(rp1, 1 - par)

        def super_body(go, carry):
            for j in range(_RING):
                pltpu.make_async_copy(ones, shared.at[idxs[j]],
                                      sems[j]).wait()
                compute_chunk(go * _RING + j, j)
                pltpu.async_copy(ones, shared.at[idxs[j]], sems[j])

            def max_body(i, mcarry):
                m0, m1, m2, m3 = mcarry
                base = ((go - 1) * _MB + i) * (4 * _L)
                v0 = ptsn[pl.ds(base, _L)]
                v1 = ptsn[pl.ds(base + _L, _L)]
                v2 = ptsn[pl.ds(base + 2 * _L, _L)]
                v3 = ptsn[pl.ds(base + 3 * _L, _L)]
                return (jnp.maximum(m0, jnp.abs(v0)),
                        jnp.maximum(m1, jnp.abs(v1)),
                        jnp.maximum(m2, jnp.abs(v2)),
                        jnp.maximum(m3, jnp.abs(v3)))
            return lax.fori_loop(0, _MB, max_body, carry, unroll=2)

        z4 = jnp.zeros((_L,), jnp.float32)
        carry = lax.fori_loop(1, _NCHUNK // _RING, super_body,
                              (z4, z4, z4, z4))

        def max_tail(i, mcarry):
            m0, m1, m2, m3 = mcarry
            base = (_NSUP1 * _MB + i) * (4 * _L)
            v0 = ptsn[pl.ds(base, _L)]
            v1 = ptsn[pl.ds(base + _L, _L)]
            v2 = ptsn[pl.ds(base + 2 * _L, _L)]
            v3 = ptsn[pl.ds(base + 3 * _L, _L)]
            return (jnp.maximum(m0, jnp.abs(v0)),
                    jnp.maximum(m1, jnp.abs(v1)),
                    jnp.maximum(m2, jnp.abs(v2)),
                    jnp.maximum(m3, jnp.abs(v3)))
        m0, m1, m2, m3 = lax.fori_loop(0, _NVEC // 4 - _NSUP1 * _MB,
                                       max_tail, carry, unroll=2)
        maxv[...] = jnp.maximum(jnp.maximum(m0, m1), jnp.maximum(m2, m3))
        pltpu.sync_copy(
            maxv,
            shared.at[pl.ds(_MAXOFF + (1 - par) * _NS * _L + s * _L, _L)])

        for j in range(_RING):
            pltpu.make_async_copy(ones, shared.at[idxs[j]], sems[j]).wait()

        rp2 = jnp.where(r + 2 < _ROUNDS, r + 2, r + 2 - _ROUNDS)
        stage(rp2, par)

        plsc.subcore_barrier()

        sl = pl.ds(gbase + s * _GSLICE, _GSLICE)
        pltpu.sync_copy(shared.at[sl], out_hbm.at[b, pl.ds(s * _GSLICE,
                                                           _GSLICE)])
        pltpu.sync_copy(zeros, shared.at[sl])

    stage(jnp.int32(0), 0)
    stage(jnp.int32(1), 1)
    local_max(jnp.int32(0), 0)
    plsc.subcore_barrier()

    def round2_body(k, _):
        do_round(2 * k, 0)
        do_round(2 * k + 1, 1)
        return 0
    lax.fori_loop(0, _ROUNDS // 2, round2_body, 0)


_occupancy = pl.kernel(
    _body,
    out_type=jax.ShapeDtypeStruct((_B, _GRID), jnp.float32),
    mesh=plsc.VectorSubcoreMesh(
        core_axis_name="c", subcore_axis_name="s",
        num_cores=_NC, num_subcores=_NS),
    compiler_params=pltpu.CompilerParams(needs_layout_passes=False),
    scratch_types=[
        pltpu.VMEM((_FPT,), jnp.float32),
        pltpu.VMEM((_FPT,), jnp.float32),
        pltpu.VMEM((_CHUNK,), jnp.int32),
        pltpu.VMEM((_CHUNK,), jnp.int32),
        pltpu.VMEM((_CHUNK,), jnp.int32),
        pltpu.VMEM((_CHUNK,), jnp.int32),
        pltpu.VMEM((_CHUNK,), jnp.float32),
        pltpu.VMEM((_GSLICE,), jnp.float32),
        pltpu.VMEM((_L,), jnp.float32),
        pltpu.VMEM((_NS * _L,), jnp.float32),
        pltpu.VMEM_SHARED((2 * _GRID + 2 * _NS * _L,), jnp.float32),
        pltpu.SemaphoreType.DMA,
        pltpu.SemaphoreType.DMA,
        pltpu.SemaphoreType.DMA,
        pltpu.SemaphoreType.DMA,
        pltpu.SemaphoreType.DMA,
        pltpu.SemaphoreType.DMA,
    ],
)


def kernel(input):
    return _occupancy(input.transpose(0, 2, 1).reshape(-1))


# speedup vs baseline: 1.0261x; 1.0261x over previous
import jax
import jax.numpy as jnp
from jax import lax
from jax.experimental import pallas as pl
from jax.experimental.pallas import tpu as pltpu
from jax.experimental.pallas import tpu_sc as plsc

_NB = 64
_GRID = _NB * _NB * _NB
_B = 16
_P = 131072
_NC = 2
_NS = 16
_L = 16
_ROUNDS = _B // _NC
_PPT = _P // _NS
_FPT = _PPT * 3
_NVEC = _FPT // _L
_GSLICE = _GRID // _NS
_CHUNK = 128
_NCHUNK = _PPT // _CHUNK
_RING = 4
_MAXOFF = 2 * _GRID
_NSUP1 = _NCHUNK // _RING - 1
_MB = (_NVEC // 4) // _NSUP1


def _body(x_hbm, out_hbm, pts0, pts1, idx0, idx1, idx2, idx3, ones, zeros,
          maxv, allmax, shared, sem0, sem1, sem2, sem3, psem0, psem1):
    idxs = (idx0, idx1, idx2, idx3)
    sems = (sem0, sem1, sem2, sem3)
    ptsb = (pts0, pts1)
    psems = (psem0, psem1)
    c = lax.axis_index("c")
    s = lax.axis_index("s")

    for k in range(_CHUNK // _L):
        ones[pl.ds(k * _L, _L)] = jnp.ones((_L,), jnp.float32)

    def zero_body(i, _):
        zeros[pl.ds(i * _L, _L)] = jnp.zeros((_L,), jnp.float32)
        return 0
    lax.fori_loop(0, _GSLICE // _L, zero_body, 0)

    pltpu.sync_copy(zeros, shared.at[pl.ds(s * _GSLICE, _GSLICE)])
    pltpu.sync_copy(zeros, shared.at[pl.ds(_GRID + s * _GSLICE, _GSLICE)])

    def stage(r, par):
        b = c * _ROUNDS + r
        for q in range(3):
            pltpu.async_copy(
                x_hbm.at[pl.ds((3 * b + q) * _P + s * _PPT, _PPT)],
                ptsb[par].at[pl.ds(q * _PPT, _PPT)], psems[par])

    def stage_wait(r, par):
        b = c * _ROUNDS + r
        for q in range(3):
            pltpu.make_async_copy(
                x_hbm.at[pl.ds((3 * b + q) * _P + s * _PPT, _PPT)],
                ptsb[par].at[pl.ds(q * _PPT, _PPT)], psems[par]).wait()

    def local_max(r, par):
        stage_wait(r, par)
        pts = ptsb[par]

        def max_body(i, carry):
            m0, m1, m2, m3 = carry
            base = i * (4 * _L)
            v0 = pts[pl.ds(base, _L)]
            v1 = pts[pl.ds(base + _L, _L)]
            v2 = pts[pl.ds(base + 2 * _L, _L)]
            v3 = pts[pl.ds(base + 3 * _L, _L)]
            return (jnp.maximum(m0, jnp.abs(v0)),
                    jnp.maximum(m1, jnp.abs(v1)),
                    jnp.maximum(m2, jnp.abs(v2)),
                    jnp.maximum(m3, jnp.abs(v3)))
        z4 = jnp.zeros((_L,), jnp.float32)
        m0, m1, m2, m3 = lax.fori_loop(0, _NVEC // 4, max_body,
                                       (z4, z4, z4, z4), unroll=2)
        maxv[...] = jnp.maximum(jnp.maximum(m0, m1), jnp.maximum(m2, m3))
        pltpu.sync_copy(
            maxv,
            shared.at[pl.ds(_MAXOFF + par * _NS * _L + s * _L, _L)])

    def do_round(r, par):
        b = c * _ROUNDS + r
        pts = ptsb[par]
        ptsn = ptsb[1 - par]
        gbase = par * _GRID

        pltpu.sync_copy(
            shared.at[pl.ds(_MAXOFF + par * _NS * _L, _NS * _L)], allmax)

        def gmax_body(i, mm):
            return jnp.maximum(mm, allmax[pl.ds(i * _L, _L)])
        mm = lax.fori_loop(0, _NS, gmax_body, jnp.zeros((_L,), jnp.float32))
        gmax = mm[0]
        for i in range(1, _L):
            gmax = jnp.maximum(gmax, mm[i])
        thick = jnp.maximum(2.0 * gmax, 1e-5)
        scale_v = jnp.full((_L,), 64.0, jnp.float32) / (
            jnp.zeros((_L,), jnp.float32) + thick)
        scale = scale_v[0]

        def compute_chunk(g, j):
            for v in range(_CHUNK // _L):
                pb = g * _CHUNK + v * _L
                x = pts[pl.ds(pb, _L)]
                y = pts[pl.ds(_PPT + pb, _L)]
                z = pts[pl.ds(2 * _PPT + pb, _L)]
                cx = ((x + gmax) * scale).astype(jnp.int32)
                cy = ((y + gmax) * scale).astype(jnp.int32)
                cz = ((z + gmax) * scale).astype(jnp.int32)
                cx = jnp.clip(cx, 0, _NB - 1)
                cy = jnp.clip(cy, 0, _NB - 1)
                cz = jnp.clip(cz, 0, _NB - 1)
                flat = (cx * _NB + cy) * _NB + (cz + gbase)
                idxs[j][pl.ds(v * _L, _L)] = flat

        for j in range(_RING):
            compute_chunk(jnp.int32(j), j)
            pltpu.async_copy(ones, shared.at[idxs[j]], sems[j])

        rp1 = jnp.where(r + 1 < _ROUNDS, r + 1, r + 1 - _ROUNDS)
        stage_wait(rp1, 1 - par)

        def super_body(go, carry):
            for j in range(_RING):
                pltpu.make_async_copy(ones, shared.at[idxs[j]],
                                      sems[j]).wait()
                compute_chunk(go * _RING + j, j)
                pltpu.async_copy(ones, shared.at[idxs[j]], sems[j])

            def max_body(i, mcarry):
                m0, m1, m2, m3 = mcarry
                base = ((go - 1) * _MB + i) * (4 * _L)
                v0 = ptsn[pl.ds(base, _L)]
                v1 = ptsn[pl.ds(base + _L, _L)]
                v2 = ptsn[pl.ds(base + 2 * _L, _L)]
                v3 = ptsn[pl.ds(base + 3 * _L, _L)]
                return (jnp.maximum(m0, jnp.abs(v0)),
                        jnp.maximum(m1, jnp.abs(v1)),
                        jnp.maximum(m2, jnp.abs(v2)),
                        jnp.maximum(m3, jnp.abs(v3)))
            return lax.fori_loop(0, _MB, max_body, carry, unroll=2)

        z4 = jnp.zeros((_L,), jnp.float32)
        carry = lax.fori_loop(1, _NCHUNK // _RING, super_body,
                              (z4, z4, z4, z4))

        def max_tail(i, mcarry):
            m0, m1, m2, m3 = mcarry
            base = (_NSUP1 * _MB + i) * (4 * _L)
            v0 = ptsn[pl.ds(base, _L)]
            v1 = ptsn[pl.ds(base + _L, _L)]
            v2 = ptsn[pl.ds(base + 2 * _L, _L)]
            v3 = ptsn[pl.ds(base + 3 * _L, _L)]
            return (jnp.maximum(m0, jnp.abs(v0)),
                    jnp.maximum(m1, jnp.abs(v1)),
                    jnp.maximum(m2, jnp.abs(v2)),
                    jnp.maximum(m3, jnp.abs(v3)))
        m0, m1, m2, m3 = lax.fori_loop(0, _NVEC // 4 - _NSUP1 * _MB,
                                       max_tail, carry, unroll=2)
        maxv[...] = jnp.maximum(jnp.maximum(m0, m1), jnp.maximum(m2, m3))
        pltpu.sync_copy(
            maxv,
            shared.at[pl.ds(_MAXOFF + (1 - par) * _NS * _L + s * _L, _L)])

        for j in range(_RING):
            pltpu.make_async_copy(ones, shared.at[idxs[j]], sems[j]).wait()

        rp2 = jnp.where(r + 2 < _ROUNDS, r + 2, r + 2 - _ROUNDS)
        stage(rp2, par)

        plsc.subcore_barrier()

        sl = pl.ds(gbase + s * _GSLICE, _GSLICE)
        pltpu.sync_copy(shared.at[sl], out_hbm.at[b, pl.ds(s * _GSLICE,
                                                           _GSLICE)])
        pltpu.sync_copy(zeros, shared.at[sl])

    stage(jnp.int32(0), 0)
    stage(jnp.int32(1), 1)
    local_max(jnp.int32(0), 0)
    plsc.subcore_barrier()

    def round2_body(k, _):
        do_round(2 * k, 0)
        do_round(2 * k + 1, 1)
        return 0
    lax.fori_loop(0, _ROUNDS // 2, round2_body, 0)


_occupancy = pl.kernel(
    _body,
    out_type=jax.ShapeDtypeStruct((_B, _GRID), jnp.float32),
    mesh=plsc.VectorSubcoreMesh(
        core_axis_name="c", subcore_axis_name="s",
        num_cores=_NC, num_subcores=_NS),
    compiler_params=pltpu.CompilerParams(needs_layout_passes=False),
    scratch_types=[
        pltpu.VMEM((_FPT,), jnp.float32),
        pltpu.VMEM((_FPT,), jnp.float32),
        pltpu.VMEM((_CHUNK,), jnp.int32),
        pltpu.VMEM((_CHUNK,), jnp.int32),
        pltpu.VMEM((_CHUNK,), jnp.int32),
        pltpu.VMEM((_CHUNK,), jnp.int32),
        pltpu.VMEM((_CHUNK,), jnp.float32),
        pltpu.VMEM((_GSLICE,), jnp.float32),
        pltpu.VMEM((_L,), jnp.float32),
        pltpu.VMEM((_NS * _L,), jnp.float32),
        pltpu.VMEM_SHARED((2 * _GRID + 2 * _NS * _L,), jnp.float32),
        pltpu.SemaphoreType.DMA,
        pltpu.SemaphoreType.DMA,
        pltpu.SemaphoreType.DMA,
        pltpu.SemaphoreType.DMA,
        pltpu.SemaphoreType.DMA,
        pltpu.SemaphoreType.DMA,
    ],
)


def kernel(input):
    return _occupancy(input.transpose(0, 2, 1).reshape(-1))
